# Initial kernel scaffold; baseline (speedup 1.0000x reference)
import jax
import jax.numpy as jnp
from jax.experimental import pallas as pl


def kernel(adj, x, W_gcn, b_gcn, W_lin, gamma, beta):
    raise NotImplementedError("write your pallas kernel here")


# speedup vs baseline: 17.4767x; 17.4767x over previous
import functools

import jax
import jax.numpy as jnp
from jax import lax
from jax.experimental import pallas as pl
from jax.experimental.pallas import tpu as pltpu
from jax.experimental.pallas import tpu_sc as plsc

N = 10000
E = 320000
D = 128
NC = 2
NS = 16
NW = NC * NS
CH = 128
CPW = -(-E // (NW * CH))
EP = NW * CPW * CH
NPAD = N + 8
NSP = 10240
RWB = NSP // NS
NB = RWB // CH

_mesh = plsc.VectorSubcoreMesh(core_axis_name="c", subcore_axis_name="s")


@functools.partial(
    pl.kernel,
    out_type=jax.ShapeDtypeStruct((NC, N), jnp.float32),
    mesh=_mesh,
    scratch_types=[
        pltpu.VMEM((CPW, CH), jnp.int32),
        pltpu.VMEM((CH,), jnp.float32),
        pltpu.VMEM((N,), jnp.float32),
        pltpu.VMEM_SHARED((NPAD,), jnp.float32),
    ],
)
def _deg_kernel(col_hbm, degp_hbm, cidx_v, ones_v, stage_v, deg_sh):
    cid = lax.axis_index("c")
    sid = lax.axis_index("s")
    wid = sid * NC + cid
    ones16 = jnp.ones((16,), jnp.float32)
    for j in range(CH // 16):
        ones_v[pl.ds(16 * j, 16)] = ones16

    @pl.when(sid == 0)
    def _zero():
        def zbody(i, carry):
            stage_v[pl.ds(i * 16, 16)] = jnp.zeros((16,), jnp.float32)
            return carry
        lax.fori_loop(0, N // 16, zbody, 0)
        pltpu.sync_copy(stage_v, deg_sh.at[pl.ds(0, N)])

    pltpu.sync_copy(col_hbm.at[wid], cidx_v)
    plsc.subcore_barrier()

    def body(c, carry):
        pltpu.sync_copy(ones_v, deg_sh.at[cidx_v.at[c]], add=True)
        return carry
    lax.fori_loop(0, CPW, body, 0)
    plsc.subcore_barrier()

    @pl.when(sid == 0)
    def _writeback():
        pltpu.sync_copy(deg_sh.at[pl.ds(0, N)], stage_v)
        pltpu.sync_copy(stage_v, degp_hbm.at[cid])


@functools.partial(
    pl.kernel,
    out_type=jax.ShapeDtypeStruct((NC * NSP, D), jnp.float32),
    mesh=_mesh,
    scratch_types=[
        pltpu.VMEM((CPW, CH), jnp.int32),
        pltpu.VMEM((CPW, CH), jnp.int32),
        pltpu.VMEM((CH, D), jnp.float32),
        pltpu.VMEM_SHARED((NSP, D), jnp.float32),
        pltpu.SemaphoreType.DMA,
    ],
)
def _scatter_kernel(row_hbm, col_hbm, hs_hbm, accp_hbm,
                    ridx_v, cidx_v, rows_v, acc_sh, sem):
    cid = lax.axis_index("c")
    sid = lax.axis_index("s")
    wid = sid * NC + cid

    zrow = jnp.zeros((16,), jnp.float32)

    def zbody(i, carry):
        for j in range(D // 16):
            rows_v[i, pl.ds(16 * j, 16)] = zrow
        return carry
    lax.fori_loop(0, CH, zbody, 0)
    zbase = sid * RWB
    for k in range(NB):
        pltpu.sync_copy(rows_v, acc_sh.at[pl.ds(zbase + k * CH, CH)])

    pltpu.sync_copy(row_hbm.at[wid], ridx_v)
    pltpu.sync_copy(col_hbm.at[wid], cidx_v)
    plsc.subcore_barrier()

    def body(c, carry):
        pltpu.async_copy(hs_hbm.at[ridx_v.at[c]], rows_v, sem).wait()
        pltpu.sync_copy(rows_v, acc_sh.at[cidx_v.at[c]], add=True)
        return carry
    lax.fori_loop(0, CPW, body, 0)
    plsc.subcore_barrier()

    obase = cid * NSP + sid * RWB
    for k in range(NB):
        pltpu.sync_copy(acc_sh.at[pl.ds(zbase + k * CH, CH)], rows_v)
        pltpu.sync_copy(rows_v, accp_hbm.at[pl.ds(obase + k * CH, CH)])


BLK = 1000


def _mm_body(x_ref, wg_ref, wl_ref, d0_ref, d1_ref, hs_ref, x2_ref):
    xb = x_ref[...]
    dinv = lax.rsqrt(d0_ref[...] + d1_ref[...] + 1.0)
    h = lax.dot_general(xb, wg_ref[...], (((1,), (1,)), ((), ())),
                        preferred_element_type=jnp.float32)
    hs_ref[...] = h * dinv
    x2_ref[...] = lax.dot_general(xb, wl_ref[...], (((1,), (1,)), ((), ())),
                                  preferred_element_type=jnp.float32)


_mm_call = pl.pallas_call(
    _mm_body,
    grid=(N // BLK,),
    in_specs=[
        pl.BlockSpec((BLK, D), lambda i: (i, 0)),
        pl.BlockSpec((D, D), lambda i: (0, 0)),
        pl.BlockSpec((D, D), lambda i: (0, 0)),
        pl.BlockSpec((BLK, 1), lambda i: (i, 0)),
        pl.BlockSpec((BLK, 1), lambda i: (i, 0)),
    ],
    out_specs=(
        pl.BlockSpec((BLK, D), lambda i: (i, 0)),
        pl.BlockSpec((BLK, D), lambda i: (i, 0)),
    ),
    out_shape=(
        jax.ShapeDtypeStruct((N, D), jnp.float32),
        jax.ShapeDtypeStruct((N, D), jnp.float32),
    ),
)


def _fin_body(a0_ref, a1_ref, hs_ref, x2_ref, d0_ref, d1_ref,
              b_ref, g_ref, be_ref, out_ref):
    dinv = lax.rsqrt(d0_ref[...] + d1_ref[...] + 1.0)
    z = (dinv * (a0_ref[...] + a1_ref[...] + hs_ref[...])
         + b_ref[...] + x2_ref[...] + 1e-6)
    mu = jnp.mean(z, axis=1, keepdims=True)
    zc = z - mu
    var = jnp.mean(zc * zc, axis=1, keepdims=True)
    out_ref[...] = zc * lax.rsqrt(var + 1e-5) * g_ref[...] + be_ref[...]


_fin_call = pl.pallas_call(
    _fin_body,
    grid=(N // BLK,),
    in_specs=[
        pl.BlockSpec((BLK, D), lambda i: (i, 0)),
        pl.BlockSpec((BLK, D), lambda i: (i, 0)),
        pl.BlockSpec((BLK, D), lambda i: (i, 0)),
        pl.BlockSpec((BLK, D), lambda i: (i, 0)),
        pl.BlockSpec((BLK, 1), lambda i: (i, 0)),
        pl.BlockSpec((BLK, 1), lambda i: (i, 0)),
        pl.BlockSpec((1, D), lambda i: (0, 0)),
        pl.BlockSpec((1, D), lambda i: (0, 0)),
        pl.BlockSpec((1, D), lambda i: (0, 0)),
    ],
    out_specs=pl.BlockSpec((BLK, D), lambda i: (i, 0)),
    out_shape=jax.ShapeDtypeStruct((N, D), jnp.float32),
)


def kernel(adj, x, W_gcn, b_gcn, W_lin, gamma, beta):
    adj = adj.astype(jnp.int32)
    pad = EP - E
    rowp = jnp.concatenate(
        [adj[0], jnp.zeros((pad,), jnp.int32)]).reshape(NW, CPW, CH)
    colp = jnp.concatenate(
        [adj[1], jnp.full((pad,), N, jnp.int32)]).reshape(NW, CPW, CH)
    degp = _deg_kernel(colp)
    d0 = degp[0].reshape(N, 1)
    d1 = degp[1].reshape(N, 1)
    hs, x2 = _mm_call(x, W_gcn, W_lin, d0, d1)
    accp = _scatter_kernel(rowp, colp, hs)
    out = _fin_call(accp[:N], accp[NSP:NSP + N], hs, x2, d0, d1,
                    b_gcn.reshape(1, D), gamma.reshape(1, D),
                    beta.reshape(1, D))
    return out
